# speedup vs baseline: 4.8858x; 4.8858x over previous
import functools
import math

import jax
import jax.numpy as jnp
from jax import lax
from jax.experimental import pallas as pl
from jax.experimental.pallas import tpu as pltpu
from jax.experimental.pallas import tpu_sc as plsc

VOCAB = 100000
EMB = 128
SCALE = math.sqrt(EMB)

NC = 2
NS = 16
NW = NC * NS

CHUNK = 128


def _scale_body(emb_ref, out_ref):
    out_ref[...] = emb_ref[...] * SCALE


def _scale_table(embeddings):
    rows = embeddings.shape[0]
    block = 5000
    grid = rows // block
    return pl.pallas_call(
        _scale_body,
        grid=(grid,),
        in_specs=[pl.BlockSpec((block, EMB), lambda i: (i, 0))],
        out_specs=pl.BlockSpec((block, EMB), lambda i: (i, 0)),
        out_shape=jax.ShapeDtypeStruct((rows, EMB), jnp.float32),
    )(embeddings)


def _make_gather(n_tokens):
    per_w = n_tokens // NW
    n_chunks = per_w // CHUNK
    mesh = plsc.VectorSubcoreMesh(core_axis_name="c", subcore_axis_name="s")

    @functools.partial(
        pl.kernel,
        mesh=mesh,
        out_type=jax.ShapeDtypeStruct((n_tokens, EMB), jnp.float32),
        scratch_types=[
            pltpu.VMEM((CHUNK,), jnp.int32),
            pltpu.VMEM((CHUNK, EMB), jnp.float32),
            pltpu.SemaphoreType.DMA,
        ],
    )
    def gather_kernel(tok_hbm, table_hbm, out_hbm, idx_v, rows_v, sem):
        wid = lax.axis_index("s") * NC + lax.axis_index("c")
        base = wid * per_w

        def body(g, carry):
            off = base + g * CHUNK
            pltpu.sync_copy(tok_hbm.at[pl.ds(off, CHUNK)], idx_v)
            pltpu.async_copy(table_hbm.at[idx_v], rows_v, sem).wait()
            pltpu.sync_copy(rows_v, out_hbm.at[pl.ds(off, CHUNK)])
            return carry

        lax.fori_loop(0, n_chunks, body, 0)

    return gather_kernel


def kernel(tokens, embeddings):
    b, t = tokens.shape
    flat = tokens.reshape(b * t).astype(jnp.int32)
    table = _scale_table(embeddings)
    out = _make_gather(b * t)(flat, table)
    return out.reshape(b, t, EMB)


# speedup vs baseline: 8.2946x; 1.6977x over previous
import functools
import math

import jax
import jax.numpy as jnp
from jax import lax
from jax.experimental import pallas as pl
from jax.experimental.pallas import tpu as pltpu
from jax.experimental.pallas import tpu_sc as plsc

VOCAB = 100000
EMB = 128
SCALE = math.sqrt(EMB)

NC = 2
NS = 16
NW = NC * NS

CHUNK = 128


def _scale_body(emb_ref, out_ref):
    out_ref[...] = emb_ref[...] * SCALE


def _scale_table(embeddings):
    rows = embeddings.shape[0]
    block = 5000
    grid = rows // block
    return pl.pallas_call(
        _scale_body,
        grid=(grid,),
        in_specs=[pl.BlockSpec((block, EMB), lambda i: (i, 0))],
        out_specs=pl.BlockSpec((block, EMB), lambda i: (i, 0)),
        out_shape=jax.ShapeDtypeStruct((rows, EMB), jnp.float32),
    )(embeddings)


STG = 2
ROWS = STG * CHUNK


def _make_gather(n_tokens):
    per_w = n_tokens // NW
    n_chunk_rows = per_w // CHUNK
    n_stages = n_chunk_rows // STG
    mesh = plsc.VectorSubcoreMesh(core_axis_name="c", subcore_axis_name="s")

    @functools.partial(
        pl.kernel,
        mesh=mesh,
        out_type=jax.ShapeDtypeStruct((n_tokens, EMB), jnp.float32),
        scratch_types=[
            pltpu.VMEM((n_chunk_rows, CHUNK), jnp.int32),
            pltpu.VMEM((ROWS, EMB), jnp.float32),
            pltpu.VMEM((ROWS, EMB), jnp.float32),
            pltpu.SemaphoreType.DMA,
            pltpu.SemaphoreType.DMA,
            pltpu.SemaphoreType.DMA,
            pltpu.SemaphoreType.DMA,
        ],
    )
    def gather_kernel(tok_hbm, table_hbm, out_hbm, idx_v,
                      rows_a, rows_b, gsem_a, gsem_b, wsem_a, wsem_b):
        wid = lax.axis_index("s") * NC + lax.axis_index("c")
        row_base = wid * per_w

        pltpu.sync_copy(tok_hbm.at[pl.ds(wid * n_chunk_rows, n_chunk_rows)],
                        idx_v)

        def g_start(s, rows, gsem):
            for j in range(STG):
                pltpu.async_copy(table_hbm.at[idx_v.at[s * STG + j]],
                                 rows.at[pl.ds(j * CHUNK, CHUNK)], gsem)

        def g_wait(rows, gsem):
            for j in range(STG):
                pltpu.make_async_copy(table_hbm.at[idx_v.at[0]],
                                      rows.at[pl.ds(j * CHUNK, CHUNK)],
                                      gsem).wait()

        def w_start(s, rows, wsem):
            pltpu.async_copy(rows, out_hbm.at[pl.ds(row_base + s * ROWS, ROWS)],
                             wsem)

        def w_wait(rows, wsem):
            pltpu.make_async_copy(rows, out_hbm.at[pl.ds(row_base, ROWS)],
                                  wsem).wait()

        g_start(0, rows_a, gsem_a)

        def phase(s, r_x, g_x, w_x, r_y, g_y, w_y):
            g_wait(r_x, g_x)
            w_start(s, r_x, w_x)

            @pl.when(s + 1 < n_stages)
            def _():
                @pl.when(s > 0)
                def _():
                    w_wait(r_y, w_y)

                g_start(s + 1, r_y, g_y)

        def body(i, carry):
            s = i * 2
            phase(s, rows_a, gsem_a, wsem_a, rows_b, gsem_b, wsem_b)
            phase(s + 1, rows_b, gsem_b, wsem_b, rows_a, gsem_a, wsem_a)
            return carry

        lax.fori_loop(0, n_stages // 2, body, 0)
        w_wait(rows_a, wsem_a)
        w_wait(rows_b, wsem_b)

    return gather_kernel


def kernel(tokens, embeddings):
    b, t = tokens.shape
    flat = tokens.reshape(b * t // CHUNK, CHUNK).astype(jnp.int32)
    table = _scale_table(embeddings)
    out = _make_gather(b * t)(flat, table)
    return out.reshape(b, t, EMB)


# speedup vs baseline: 8.3306x; 1.0043x over previous
import functools
import math

import jax
import jax.numpy as jnp
from jax import lax
from jax.experimental import pallas as pl
from jax.experimental.pallas import tpu as pltpu
from jax.experimental.pallas import tpu_sc as plsc

VOCAB = 100000
EMB = 128
SCALE = math.sqrt(EMB)

NC = 2
NS = 16
NW = NC * NS

CHUNK = 128


def _scale_body(emb_ref, out_ref):
    out_ref[...] = emb_ref[...] * SCALE


def _scale_table(embeddings):
    rows = embeddings.shape[0]
    block = 5000
    grid = rows // block
    return pl.pallas_call(
        _scale_body,
        grid=(grid,),
        in_specs=[pl.BlockSpec((block, EMB), lambda i: (i, 0))],
        out_specs=pl.BlockSpec((block, EMB), lambda i: (i, 0)),
        out_shape=jax.ShapeDtypeStruct((rows, EMB), jnp.float32),
    )(embeddings)


NBUF = 4


def _make_gather(n_tokens):
    per_w = n_tokens // NW
    n_stages = per_w // CHUNK
    mesh = plsc.VectorSubcoreMesh(core_axis_name="c", subcore_axis_name="s")

    @functools.partial(
        pl.kernel,
        mesh=mesh,
        out_type=jax.ShapeDtypeStruct((n_tokens, EMB), jnp.float32),
        scratch_types=[
            pltpu.VMEM((n_stages, CHUNK), jnp.int32),
        ] + [pltpu.VMEM((CHUNK, EMB), jnp.float32)] * NBUF
          + [pltpu.SemaphoreType.DMA] * (2 * NBUF),
    )
    def gather_kernel(tok_hbm, table_hbm, out_hbm, idx_v, *bufs):
        rows = bufs[:NBUF]
        gsem = bufs[NBUF:2 * NBUF]
        wsem = bufs[2 * NBUF:]
        wid = lax.axis_index("s") * NC + lax.axis_index("c")
        row_base = wid * per_w

        pltpu.sync_copy(tok_hbm.at[pl.ds(wid * n_stages, n_stages)], idx_v)

        def g_start(s, b):
            pltpu.async_copy(table_hbm.at[idx_v.at[s]], rows[b], gsem[b])

        def g_wait(b):
            pltpu.make_async_copy(table_hbm.at[idx_v.at[0]], rows[b],
                                  gsem[b]).wait()

        def w_start(s, b):
            pltpu.async_copy(rows[b],
                             out_hbm.at[pl.ds(row_base + s * CHUNK, CHUNK)],
                             wsem[b])

        def w_wait(b):
            pltpu.make_async_copy(rows[b], out_hbm.at[pl.ds(row_base, CHUNK)],
                                  wsem[b]).wait()

        for s0 in range(NBUF - 1):
            g_start(s0, s0)

        def phase(s, b):
            g_wait(b)
            w_start(s, b)
            nxt = (b + NBUF - 1) % NBUF

            @pl.when(s + NBUF - 1 < n_stages)
            def _():
                @pl.when(s >= 1)
                def _():
                    w_wait(nxt)
                g_start(s + NBUF - 1, nxt)

        def body(i, carry):
            s = i * NBUF
            for b in range(NBUF):
                phase(s + b, b)
            return carry

        lax.fori_loop(0, n_stages // NBUF, body, 0)
        for b in range(NBUF):
            w_wait(b)

    return gather_kernel


def kernel(tokens, embeddings):
    b, t = tokens.shape
    flat = tokens.reshape(b * t // CHUNK, CHUNK).astype(jnp.int32)
    table = _scale_table(embeddings)
    out = _make_gather(b * t)(flat, table)
    return out.reshape(b, t, EMB)


# speedup vs baseline: 9.1506x; 1.0984x over previous
import functools
import math

import jax
import jax.numpy as jnp
from jax import lax
from jax.experimental import pallas as pl
from jax.experimental.pallas import tpu as pltpu
from jax.experimental.pallas import tpu_sc as plsc

VOCAB = 100000
EMB = 128
SCALE = math.sqrt(EMB)

NC = 2
NS = 16
NW = NC * NS

CHUNK = 128


def _scale_body(emb_ref, out_ref):
    out_ref[...] = emb_ref[...] * SCALE


def _scale_table(embeddings):
    rows = embeddings.shape[0]
    block = 5000
    grid = rows // block
    return pl.pallas_call(
        _scale_body,
        grid=(grid,),
        in_specs=[pl.BlockSpec((block, EMB), lambda i: (i, 0))],
        out_specs=pl.BlockSpec((block, EMB), lambda i: (i, 0)),
        out_shape=jax.ShapeDtypeStruct((rows, EMB), jnp.float32),
    )(embeddings)


NBUF = 4


def _make_gather(n_tokens):
    per_w = n_tokens // NW
    n_stages = per_w // CHUNK
    mesh = plsc.VectorSubcoreMesh(core_axis_name="c", subcore_axis_name="s")

    @functools.partial(
        pl.kernel,
        mesh=mesh,
        out_type=jax.ShapeDtypeStruct((n_tokens, EMB), jnp.float32),
        scratch_types=[
            pltpu.VMEM((n_stages, CHUNK), jnp.int32),
        ] + [pltpu.VMEM((CHUNK, EMB), jnp.float32)] * NBUF
          + [pltpu.SemaphoreType.DMA] * (2 * NBUF),
    )
    def gather_kernel(tok_hbm, table_hbm, out_hbm, idx_v, *bufs):
        rows = bufs[:NBUF]
        gsem = bufs[NBUF:2 * NBUF]
        wsem = bufs[2 * NBUF:]
        wid = lax.axis_index("s") * NC + lax.axis_index("c")
        row_base = wid * per_w

        pltpu.sync_copy(tok_hbm.at[pl.ds(wid * n_stages, n_stages)], idx_v)

        def g_start(s, b):
            pltpu.async_copy(table_hbm.at[idx_v.at[s]], rows[b], gsem[b])

        def g_wait(b):
            pltpu.make_async_copy(table_hbm.at[idx_v.at[0]], rows[b],
                                  gsem[b]).wait()

        def w_start(s, b):
            pltpu.async_copy(rows[b],
                             out_hbm.at[pl.ds(row_base + s * CHUNK, CHUNK)],
                             wsem[b])

        def w_wait(b):
            pltpu.make_async_copy(rows[b], out_hbm.at[pl.ds(row_base, CHUNK)],
                                  wsem[b]).wait()

        for s0 in range(NBUF - 1):
            g_start(s0, s0)

        def scale_rows(b):
            buf = rows[b]

            def srow(r, carry):
                for k in range(8):
                    sl = (r, pl.ds(k * 16, 16))
                    buf[sl] = buf[sl] * SCALE
                return carry

            lax.fori_loop(0, CHUNK, srow, 0)

        def phase(s, b):
            g_wait(b)
            scale_rows(b)
            w_start(s, b)
            nxt = (b + NBUF - 1) % NBUF

            @pl.when(s + NBUF - 1 < n_stages)
            def _():
                @pl.when(s >= 1)
                def _():
                    w_wait(nxt)
                g_start(s + NBUF - 1, nxt)

        def body(i, carry):
            s = i * NBUF
            for b in range(NBUF):
                phase(s + b, b)
            return carry

        lax.fori_loop(0, n_stages // NBUF, body, 0)
        for b in range(NBUF):
            w_wait(b)

    return gather_kernel


def kernel(tokens, embeddings):
    b, t = tokens.shape
    flat = tokens.reshape(b * t // CHUNK, CHUNK).astype(jnp.int32)
    out = _make_gather(b * t)(flat, embeddings)
    return out.reshape(b, t, EMB)
